# baseline (speedup 1.0000x reference)
import functools

import jax
import jax.numpy as jnp
from jax import lax
from jax.experimental import pallas as pl
from jax.experimental.pallas import tpu as pltpu
from jax.experimental.pallas import tpu_sc as plsc

_N = 320000
_D = 128
_S = 10000
_B = 128
_NC = 2
_NS = 16
_NW = _NC * _NS
_TT = 2048
_NT_TC = 48
_NTC = _NT_TC * _TT
_NB_SC = (_N - _NTC) // _B
_B_PER_W = 55
_NRING = 3
_WIN = 128
_S_PAD = 10128
_ROWS_PER_TILE = 624
_TAIL_ROWS = _S - _NS * _ROWS_PER_TILE


def _sc_segment_sum(features, sids2d, zrows):
    mesh = plsc.VectorSubcoreMesh(core_axis_name="c", subcore_axis_name="s")

    @functools.partial(
        pl.kernel,
        out_type=jax.ShapeDtypeStruct((_NC, _S, _D), jnp.float32),
        mesh=mesh,
        scratch_types=[
            pltpu.VMEM_SHARED((_S, _D), jnp.float32),
            pltpu.VMEM((_NRING, _B, _D), jnp.float32),
            pltpu.VMEM((_NRING, 1, _B), jnp.int32),
            pltpu.SemaphoreType.DMA((_NRING,)),
            pltpu.SemaphoreType.DMA((_NRING,)),
        ],
    )
    def seg_kernel(feat_hbm, idx_hbm, zero_hbm, out_hbm, acc_sh, fbuf, ibuf,
                   lsem, ssem):
        c = lax.axis_index("c")
        s = lax.axis_index("s")
        wid = c * _NS + s
        cnt = jnp.clip(_NB_SC - wid * _B_PER_W, 0, _B_PER_W)

        def _load(t, slot):
            g = wid * _B_PER_W + t
            pltpu.async_copy(feat_hbm.at[pl.ds(_NTC + g * _B, _B)],
                             fbuf.at[slot], lsem.at[slot])
            pltpu.async_copy(idx_hbm.at[g], ibuf.at[slot, 0], lsem.at[slot])

        def _wait_load(slot):
            pltpu.make_async_copy(feat_hbm.at[pl.ds(0, _B)],
                                  fbuf.at[slot], lsem.at[slot]).wait()
            pltpu.make_async_copy(idx_hbm.at[0], ibuf.at[slot, 0],
                                  lsem.at[slot]).wait()

        def _scatter(slot):
            pltpu.async_copy(fbuf.at[slot], acc_sh.at[ibuf.at[slot, 0]],
                             ssem.at[slot], add=True)

        def _wait_scatter(slot):
            pltpu.make_async_copy(fbuf.at[slot],
                                  acc_sh.at[ibuf.at[slot, 0]],
                                  ssem.at[slot]).wait()

        @pl.when(0 < cnt)
        def _():
            _load(0, 0)

        @pl.when(1 < cnt)
        def _():
            _load(1, 1)

        row0 = s * _ROWS_PER_TILE
        pltpu.sync_copy(zero_hbm.at[pl.ds(row0, _ROWS_PER_TILE)],
                        acc_sh.at[pl.ds(row0, _ROWS_PER_TILE)])

        @pl.when(s == _NS - 1)
        def _zero_tail():
            pltpu.sync_copy(
                zero_hbm.at[pl.ds(_NS * _ROWS_PER_TILE, _TAIL_ROWS)],
                acc_sh.at[pl.ds(_NS * _ROWS_PER_TILE, _TAIL_ROWS)])

        plsc.subcore_barrier()

        def _batch(t, slot):
            @pl.when(t < cnt)
            def _():
                _wait_load(slot)
                _scatter(slot)

            nslot = (slot + 2) % _NRING

            @pl.when((t >= 1) & (t - 1 < cnt))
            def _():
                _wait_scatter(nslot)

            @pl.when(t + 2 < cnt)
            def _():
                _load(t + 2, nslot)

        def _step(i, _):
            t = 3 * i
            _batch(t, 0)
            _batch(t + 1, 1)
            _batch(t + 2, 2)
            return 0

        lax.fori_loop(0, 19, _step, 0)
        plsc.subcore_barrier()

        pltpu.sync_copy(acc_sh.at[pl.ds(row0, _ROWS_PER_TILE)],
                        out_hbm.at[c, pl.ds(row0, _ROWS_PER_TILE)])

        @pl.when(s == _NS - 1)
        def _copy_tail():
            pltpu.sync_copy(acc_sh.at[pl.ds(_NS * _ROWS_PER_TILE, _TAIL_ROWS)],
                            out_hbm.at[c, pl.ds(_NS * _ROWS_PER_TILE,
                                                _TAIL_ROWS)])

    return seg_kernel(features, sids2d, zrows)


def _tc_seg_body(sid_ref, x_ref, o_ref):
    i = pl.program_id(0)

    @pl.when(i == 0)
    def _init():
        o_ref[...] = jnp.zeros_like(o_ref)

    sids = sid_ref[0, 0, :]
    x = x_ref[...]
    base = (sids[0] // 8) * 8
    nwin = (sids[_TT - 1] - base) // _WIN + 1

    def _window(j, _):
        w0 = pl.multiple_of(base + j * _WIN, 8)
        seg = w0 + lax.broadcasted_iota(jnp.int32, (_WIN, _TT), 0)
        m = (seg == sids[None, :]).astype(jnp.float32)
        o_ref[pl.ds(w0, _WIN), :] += jnp.dot(
            m, x, preferred_element_type=jnp.float32)
        return 0

    lax.fori_loop(0, nwin, _window, 0)


_tc_segsum = pl.pallas_call(
    _tc_seg_body,
    grid=(_NT_TC,),
    in_specs=[
        pl.BlockSpec((1, 1, _TT), lambda i: (i, 0, 0)),
        pl.BlockSpec((_TT, _D), lambda i: (i, 0)),
    ],
    out_specs=pl.BlockSpec((_S_PAD, _D), lambda i: (0, 0)),
    out_shape=jax.ShapeDtypeStruct((_S_PAD, _D), jnp.float32),
)


def _mm_body(p_ref, ptc_ref, w_ref, o_ref):
    o_ref[...] = jnp.dot(p_ref[0] + p_ref[1] + ptc_ref[...], w_ref[...],
                         preferred_element_type=jnp.float32)


_mm = pl.pallas_call(
    _mm_body,
    grid=(10,),
    in_specs=[
        pl.BlockSpec((_NC, _S // 10, _D), lambda i: (0, i, 0)),
        pl.BlockSpec((_S // 10, _D), lambda i: (i, 0)),
        pl.BlockSpec((_D, _D), lambda i: (0, 0)),
    ],
    out_specs=pl.BlockSpec((_S // 10, _D), lambda i: (i, 0)),
    out_shape=jax.ShapeDtypeStruct((_S, _D), jnp.float32),
)


def kernel(features, structural_indices, n_structures, W):
    del n_structures
    sids = structural_indices.astype(jnp.int32)
    sids2d = sids[_NTC:].reshape(_NB_SC, _B)
    sids_tc = sids[:_NTC].reshape(_NT_TC, 1, _TT)
    zrows = jnp.zeros((_S, _D), jnp.float32)
    partials = _sc_segment_sum(features, sids2d, zrows)
    partial_tc = _tc_segsum(sids_tc, features)
    return _mm(partials, partial_tc, W)


# speedup vs baseline: 1.0399x; 1.0399x over previous
import functools

import jax
import jax.numpy as jnp
from jax import lax
from jax.experimental import pallas as pl
from jax.experimental.pallas import tpu as pltpu
from jax.experimental.pallas import tpu_sc as plsc

_N = 320000
_D = 128
_S = 10000
_B = 128
_NC = 2
_NS = 16
_NW = _NC * _NS
_TT = 2048
_NT_TC = 56
_NTC = _NT_TC * _TT
_NB_SC = (_N - _NTC) // _B
_B_PER_W = 51
_NRING = 3
_WIN = 128
_S_PAD = 10128
_ROWS_PER_TILE = 624
_TAIL_ROWS = _S - _NS * _ROWS_PER_TILE


def _sc_segment_sum(features, sids2d, zrows):
    mesh = plsc.VectorSubcoreMesh(core_axis_name="c", subcore_axis_name="s")

    @functools.partial(
        pl.kernel,
        out_type=jax.ShapeDtypeStruct((_NC, _S, _D), jnp.float32),
        mesh=mesh,
        scratch_types=[
            pltpu.VMEM_SHARED((_S, _D), jnp.float32),
            pltpu.VMEM((_NRING, _B, _D), jnp.float32),
            pltpu.VMEM((_NRING, 1, _B), jnp.int32),
            pltpu.SemaphoreType.DMA((_NRING,)),
            pltpu.SemaphoreType.DMA((_NRING,)),
        ],
    )
    def seg_kernel(feat_hbm, idx_hbm, zero_hbm, out_hbm, acc_sh, fbuf, ibuf,
                   lsem, ssem):
        c = lax.axis_index("c")
        s = lax.axis_index("s")
        wid = c * _NS + s
        cnt = jnp.clip(_NB_SC - wid * _B_PER_W, 0, _B_PER_W)

        def _load(t, slot):
            g = wid * _B_PER_W + t
            pltpu.async_copy(feat_hbm.at[pl.ds(_NTC + g * _B, _B)],
                             fbuf.at[slot], lsem.at[slot])
            pltpu.async_copy(idx_hbm.at[g], ibuf.at[slot, 0], lsem.at[slot])

        def _wait_load(slot):
            pltpu.make_async_copy(feat_hbm.at[pl.ds(0, _B)],
                                  fbuf.at[slot], lsem.at[slot]).wait()
            pltpu.make_async_copy(idx_hbm.at[0], ibuf.at[slot, 0],
                                  lsem.at[slot]).wait()

        def _scatter(slot):
            pltpu.async_copy(fbuf.at[slot], acc_sh.at[ibuf.at[slot, 0]],
                             ssem.at[slot], add=True)

        def _wait_scatter(slot):
            pltpu.make_async_copy(fbuf.at[slot],
                                  acc_sh.at[ibuf.at[slot, 0]],
                                  ssem.at[slot]).wait()

        @pl.when(0 < cnt)
        def _():
            _load(0, 0)

        @pl.when(1 < cnt)
        def _():
            _load(1, 1)

        row0 = s * _ROWS_PER_TILE
        pltpu.sync_copy(zero_hbm.at[pl.ds(row0, _ROWS_PER_TILE)],
                        acc_sh.at[pl.ds(row0, _ROWS_PER_TILE)])

        @pl.when(s == _NS - 1)
        def _zero_tail():
            pltpu.sync_copy(
                zero_hbm.at[pl.ds(_NS * _ROWS_PER_TILE, _TAIL_ROWS)],
                acc_sh.at[pl.ds(_NS * _ROWS_PER_TILE, _TAIL_ROWS)])

        plsc.subcore_barrier()

        def _batch(t, slot):
            @pl.when(t < cnt)
            def _():
                _wait_load(slot)
                _scatter(slot)

            nslot = (slot + 2) % _NRING

            @pl.when((t >= 1) & (t - 1 < cnt))
            def _():
                _wait_scatter(nslot)

            @pl.when(t + 2 < cnt)
            def _():
                _load(t + 2, nslot)

        def _step(i, _):
            t = 3 * i
            _batch(t, 0)
            _batch(t + 1, 1)
            _batch(t + 2, 2)
            return 0

        lax.fori_loop(0, 18, _step, 0)
        plsc.subcore_barrier()

        pltpu.sync_copy(acc_sh.at[pl.ds(row0, _ROWS_PER_TILE)],
                        out_hbm.at[c, pl.ds(row0, _ROWS_PER_TILE)])

        @pl.when(s == _NS - 1)
        def _copy_tail():
            pltpu.sync_copy(acc_sh.at[pl.ds(_NS * _ROWS_PER_TILE, _TAIL_ROWS)],
                            out_hbm.at[c, pl.ds(_NS * _ROWS_PER_TILE,
                                                _TAIL_ROWS)])

    return seg_kernel(features, sids2d, zrows)


def _tc_seg_body(sid_ref, x_ref, o_ref):
    i = pl.program_id(0)

    @pl.when(i == 0)
    def _init():
        o_ref[...] = jnp.zeros_like(o_ref)

    sids = sid_ref[0, 0, :]
    x = x_ref[...]
    base = (sids[0] // 8) * 8
    nwin = (sids[_TT - 1] - base) // _WIN + 1

    def _window(j, _):
        w0 = pl.multiple_of(base + j * _WIN, 8)
        seg = w0 + lax.broadcasted_iota(jnp.int32, (_WIN, _TT), 0)
        m = (seg == sids[None, :]).astype(jnp.float32)
        o_ref[pl.ds(w0, _WIN), :] += jnp.dot(
            m, x, preferred_element_type=jnp.float32)
        return 0

    lax.fori_loop(0, nwin, _window, 0)


_tc_segsum = pl.pallas_call(
    _tc_seg_body,
    grid=(_NT_TC,),
    in_specs=[
        pl.BlockSpec((1, 1, _TT), lambda i: (i, 0, 0)),
        pl.BlockSpec((_TT, _D), lambda i: (i, 0)),
    ],
    out_specs=pl.BlockSpec((_S_PAD, _D), lambda i: (0, 0)),
    out_shape=jax.ShapeDtypeStruct((_S_PAD, _D), jnp.float32),
)


def _mm_body(p_ref, ptc_ref, w_ref, o_ref):
    o_ref[...] = jnp.dot(p_ref[0] + p_ref[1] + ptc_ref[...], w_ref[...],
                         preferred_element_type=jnp.float32)


_mm = pl.pallas_call(
    _mm_body,
    grid=(10,),
    in_specs=[
        pl.BlockSpec((_NC, _S // 10, _D), lambda i: (0, i, 0)),
        pl.BlockSpec((_S // 10, _D), lambda i: (i, 0)),
        pl.BlockSpec((_D, _D), lambda i: (0, 0)),
    ],
    out_specs=pl.BlockSpec((_S // 10, _D), lambda i: (i, 0)),
    out_shape=jax.ShapeDtypeStruct((_S, _D), jnp.float32),
)


def kernel(features, structural_indices, n_structures, W):
    del n_structures
    sids = structural_indices.astype(jnp.int32)
    sids2d = sids[_NTC:].reshape(_NB_SC, _B)
    sids_tc = sids[:_NTC].reshape(_NT_TC, 1, _TT)
    zrows = jnp.zeros((_S, _D), jnp.float32)
    partials = _sc_segment_sum(features, sids2d, zrows)
    partial_tc = _tc_segsum(sids_tc, features)
    return _mm(partials, partial_tc, W)


# speedup vs baseline: 1.0832x; 1.0417x over previous
import functools

import jax
import jax.numpy as jnp
from jax import lax
from jax.experimental import pallas as pl
from jax.experimental.pallas import tpu as pltpu
from jax.experimental.pallas import tpu_sc as plsc

_N = 320000
_D = 128
_S = 10000
_B = 128
_NC = 2
_NS = 16
_NW = _NC * _NS
_TT = 2048
_NT_TC = 64
_NTC = _NT_TC * _TT
_NB_SC = (_N - _NTC) // _B
_B_PER_W = 47
_NRING = 3
_WIN = 128
_S_PAD = 10128
_ROWS_PER_TILE = 624
_TAIL_ROWS = _S - _NS * _ROWS_PER_TILE


def _sc_segment_sum(features, sids2d, zrows):
    mesh = plsc.VectorSubcoreMesh(core_axis_name="c", subcore_axis_name="s")

    @functools.partial(
        pl.kernel,
        out_type=jax.ShapeDtypeStruct((_NC, _S, _D), jnp.float32),
        mesh=mesh,
        scratch_types=[
            pltpu.VMEM_SHARED((_S, _D), jnp.float32),
            pltpu.VMEM((_NRING, _B, _D), jnp.float32),
            pltpu.VMEM((_NRING, 1, _B), jnp.int32),
            pltpu.SemaphoreType.DMA((_NRING,)),
            pltpu.SemaphoreType.DMA((_NRING,)),
        ],
    )
    def seg_kernel(feat_hbm, idx_hbm, zero_hbm, out_hbm, acc_sh, fbuf, ibuf,
                   lsem, ssem):
        c = lax.axis_index("c")
        s = lax.axis_index("s")
        wid = c * _NS + s
        cnt = jnp.clip(_NB_SC - wid * _B_PER_W, 0, _B_PER_W)

        def _load(t, slot):
            g = wid * _B_PER_W + t
            pltpu.async_copy(feat_hbm.at[pl.ds(_NTC + g * _B, _B)],
                             fbuf.at[slot], lsem.at[slot])
            pltpu.async_copy(idx_hbm.at[g], ibuf.at[slot, 0], lsem.at[slot])

        def _wait_load(slot):
            pltpu.make_async_copy(feat_hbm.at[pl.ds(0, _B)],
                                  fbuf.at[slot], lsem.at[slot]).wait()
            pltpu.make_async_copy(idx_hbm.at[0], ibuf.at[slot, 0],
                                  lsem.at[slot]).wait()

        def _scatter(slot):
            pltpu.async_copy(fbuf.at[slot], acc_sh.at[ibuf.at[slot, 0]],
                             ssem.at[slot], add=True)

        def _wait_scatter(slot):
            pltpu.make_async_copy(fbuf.at[slot],
                                  acc_sh.at[ibuf.at[slot, 0]],
                                  ssem.at[slot]).wait()

        @pl.when(0 < cnt)
        def _():
            _load(0, 0)

        @pl.when(1 < cnt)
        def _():
            _load(1, 1)

        row0 = s * _ROWS_PER_TILE
        pltpu.sync_copy(zero_hbm.at[pl.ds(row0, _ROWS_PER_TILE)],
                        acc_sh.at[pl.ds(row0, _ROWS_PER_TILE)])

        @pl.when(s == _NS - 1)
        def _zero_tail():
            pltpu.sync_copy(
                zero_hbm.at[pl.ds(_NS * _ROWS_PER_TILE, _TAIL_ROWS)],
                acc_sh.at[pl.ds(_NS * _ROWS_PER_TILE, _TAIL_ROWS)])

        plsc.subcore_barrier()

        def _batch(t, slot):
            @pl.when(t < cnt)
            def _():
                _wait_load(slot)
                _scatter(slot)

            nslot = (slot + 2) % _NRING

            @pl.when((t >= 1) & (t - 1 < cnt))
            def _():
                _wait_scatter(nslot)

            @pl.when(t + 2 < cnt)
            def _():
                _load(t + 2, nslot)

        def _step(i, _):
            t = 3 * i
            _batch(t, 0)
            _batch(t + 1, 1)
            _batch(t + 2, 2)
            return 0

        lax.fori_loop(0, 17, _step, 0)
        plsc.subcore_barrier()

        pltpu.sync_copy(acc_sh.at[pl.ds(row0, _ROWS_PER_TILE)],
                        out_hbm.at[c, pl.ds(row0, _ROWS_PER_TILE)])

        @pl.when(s == _NS - 1)
        def _copy_tail():
            pltpu.sync_copy(acc_sh.at[pl.ds(_NS * _ROWS_PER_TILE, _TAIL_ROWS)],
                            out_hbm.at[c, pl.ds(_NS * _ROWS_PER_TILE,
                                                _TAIL_ROWS)])

    return seg_kernel(features, sids2d, zrows)


def _tc_seg_body(sid_ref, x_ref, o_ref):
    i = pl.program_id(0)

    @pl.when(i == 0)
    def _init():
        o_ref[...] = jnp.zeros_like(o_ref)

    sids = sid_ref[0, 0, :]
    x = x_ref[...]
    base = (sids[0] // 8) * 8
    nwin = (sids[_TT - 1] - base) // _WIN + 1

    def _window(j, _):
        w0 = pl.multiple_of(base + j * _WIN, 8)
        seg = w0 + lax.broadcasted_iota(jnp.int32, (_WIN, _TT), 0)
        m = (seg == sids[None, :]).astype(jnp.float32)
        o_ref[pl.ds(w0, _WIN), :] += jnp.dot(
            m, x, preferred_element_type=jnp.float32)
        return 0

    lax.fori_loop(0, nwin, _window, 0)


_tc_segsum = pl.pallas_call(
    _tc_seg_body,
    grid=(_NT_TC,),
    in_specs=[
        pl.BlockSpec((1, 1, _TT), lambda i: (i, 0, 0)),
        pl.BlockSpec((_TT, _D), lambda i: (i, 0)),
    ],
    out_specs=pl.BlockSpec((_S_PAD, _D), lambda i: (0, 0)),
    out_shape=jax.ShapeDtypeStruct((_S_PAD, _D), jnp.float32),
)


def _mm_body(p_ref, ptc_ref, w_ref, o_ref):
    o_ref[...] = jnp.dot(p_ref[0] + p_ref[1] + ptc_ref[...], w_ref[...],
                         preferred_element_type=jnp.float32)


_mm = pl.pallas_call(
    _mm_body,
    grid=(10,),
    in_specs=[
        pl.BlockSpec((_NC, _S // 10, _D), lambda i: (0, i, 0)),
        pl.BlockSpec((_S // 10, _D), lambda i: (i, 0)),
        pl.BlockSpec((_D, _D), lambda i: (0, 0)),
    ],
    out_specs=pl.BlockSpec((_S // 10, _D), lambda i: (i, 0)),
    out_shape=jax.ShapeDtypeStruct((_S, _D), jnp.float32),
)


def kernel(features, structural_indices, n_structures, W):
    del n_structures
    sids = structural_indices.astype(jnp.int32)
    sids2d = sids[_NTC:].reshape(_NB_SC, _B)
    sids_tc = sids[:_NTC].reshape(_NT_TC, 1, _TT)
    zrows = jnp.zeros((_S, _D), jnp.float32)
    partials = _sc_segment_sum(features, sids2d, zrows)
    partial_tc = _tc_segsum(sids_tc, features)
    return _mm(partials, partial_tc, W)
